# Initial kernel scaffold; baseline (speedup 1.0000x reference)
import jax
import jax.numpy as jnp
from jax.experimental import pallas as pl


def kernel(x, edge_index, ppi_pairs, idx, eps0, W1_0, b1_0, W2_0, b2_0, g0, be0, eps1, W1_1, b1_1, W2_1, b2_1, g1, be1, Wl, bl, Wf, bf):
    raise NotImplementedError("write your pallas kernel here")


# speedup vs baseline: 6.0364x; 6.0364x over previous
import functools

import jax
import jax.numpy as jnp
from jax import lax
from jax.experimental import pallas as pl
from jax.experimental.pallas import tpu as pltpu
from jax.experimental.pallas import tpu_sc as plsc

N = 10000
E = 320000
D = 128
H = 128
OUT = 7
P = 100000
B = 16384

NC = 2
NS = 16
NW = NC * NS

EPW = E // NW
K = 80
NCHUNK = EPW // K

ROWS_T = 624

BPW = B // NW
KP = 128
NPCHUNK = BPW // KP

_MESH = plsc.VectorSubcoreMesh(
    core_axis_name="c", subcore_axis_name="s", num_cores=NC, num_subcores=NS
)


def _agg_body(x_hbm, src_hbm, dst_hbm, zeros_hbm, p0_hbm, p1_hbm,
              srcv, dstv, rows, acc, sem):
    c = lax.axis_index("c")
    s = lax.axis_index("s")
    wid = c * NS + s

    @pl.when(s == 0)
    def _():
        pltpu.sync_copy(zeros_hbm, acc)

    plsc.subcore_barrier()

    pltpu.sync_copy(src_hbm.at[wid], srcv)
    pltpu.sync_copy(dst_hbm.at[wid], dstv)

    def chunk(i, carry):
        pltpu.async_copy(x_hbm.at[srcv.at[i]], rows, sem).wait()
        pltpu.sync_copy(rows, acc.at[dstv.at[i]], add=True)
        return carry

    lax.fori_loop(0, NCHUNK, chunk, 0)

    plsc.subcore_barrier()

    start = s * ROWS_T

    @pl.when(c == 0)
    def _():
        pltpu.sync_copy(acc.at[pl.ds(start, ROWS_T)], p0_hbm.at[pl.ds(start, ROWS_T)])

        @pl.when(s == NS - 1)
        def _():
            pltpu.sync_copy(acc.at[pl.ds(NS * ROWS_T, N - NS * ROWS_T)],
                            p0_hbm.at[pl.ds(NS * ROWS_T, N - NS * ROWS_T)])

    @pl.when(c == 1)
    def _():
        pltpu.sync_copy(acc.at[pl.ds(start, ROWS_T)], p1_hbm.at[pl.ds(start, ROWS_T)])

        @pl.when(s == NS - 1)
        def _():
            pltpu.sync_copy(acc.at[pl.ds(NS * ROWS_T, N - NS * ROWS_T)],
                            p1_hbm.at[pl.ds(NS * ROWS_T, N - NS * ROWS_T)])


_agg = functools.partial(
    pl.kernel,
    out_type=(jax.ShapeDtypeStruct((N, D), jnp.float32),
              jax.ShapeDtypeStruct((N, D), jnp.float32)),
    mesh=_MESH,
    scratch_types=[
        pltpu.VMEM((NCHUNK, K), jnp.int32),
        pltpu.VMEM((NCHUNK, K), jnp.int32),
        pltpu.VMEM((K, D), jnp.float32),
        pltpu.VMEM_SHARED((N, D), jnp.float32),
        pltpu.SemaphoreType.DMA,
    ],
)(_agg_body)


def _pair_body(h_hbm, pairs_hbm, idx_hbm, xa_hbm, xb_hbm,
               idxv, ia, ib, na, nb, rows_a, rows_b, sem):
    c = lax.axis_index("c")
    s = lax.axis_index("s")
    wid = c * NS + s
    base = wid * BPW

    for j in range(NPCHUNK):
        off = base + j * KP
        pltpu.sync_copy(idx_hbm.at[pl.ds(off, KP)], idxv)
        for l in range(KP // 16):
            v = idxv[pl.ds(l * 16, 16)]
            ia[pl.ds(l * 16, 16)] = v * 2
            ib[pl.ds(l * 16, 16)] = v * 2 + 1
        pltpu.async_copy(pairs_hbm.at[ia], na, sem).wait()
        pltpu.async_copy(pairs_hbm.at[ib], nb, sem).wait()
        pltpu.async_copy(h_hbm.at[na], rows_a, sem).wait()
        pltpu.async_copy(h_hbm.at[nb], rows_b, sem).wait()
        pltpu.sync_copy(rows_a, xa_hbm.at[pl.ds(off, KP)])
        pltpu.sync_copy(rows_b, xb_hbm.at[pl.ds(off, KP)])


_pair_gather = functools.partial(
    pl.kernel,
    out_type=(jax.ShapeDtypeStruct((B, D), jnp.float32),
              jax.ShapeDtypeStruct((B, D), jnp.float32)),
    mesh=_MESH,
    scratch_types=[
        pltpu.VMEM((KP,), jnp.int32),
        pltpu.VMEM((KP,), jnp.int32),
        pltpu.VMEM((KP,), jnp.int32),
        pltpu.VMEM((KP,), jnp.int32),
        pltpu.VMEM((KP,), jnp.int32),
        pltpu.VMEM((KP, D), jnp.float32),
        pltpu.VMEM((KP, D), jnp.float32),
        pltpu.SemaphoreType.DMA,
    ],
)(_pair_body)


def _mlp_core(x_ref, p0_ref, p1_ref, w1_ref, b1_ref, w2_ref, b2_ref,
              g_ref, be_ref, eps_ref):
    z = (1.0 + eps_ref[0, 0]) * x_ref[...] + p0_ref[...] + p1_ref[...]
    h = jnp.dot(z, w1_ref[...], preferred_element_type=jnp.float32) + b1_ref[...]
    h = jnp.maximum(h, 0.0)
    h = jnp.dot(h, w2_ref[...], preferred_element_type=jnp.float32) + b2_ref[...]
    h = jnp.maximum(h, 0.0)
    mean = jnp.mean(h, axis=0, keepdims=True)
    cent = h - mean
    var = jnp.mean(cent * cent, axis=0, keepdims=True)
    return cent * lax.rsqrt(var + 1e-5) * g_ref[...] + be_ref[...]


def _mlp_body(x_ref, p0_ref, p1_ref, w1_ref, b1_ref, w2_ref, b2_ref,
              g_ref, be_ref, eps_ref, out_ref):
    out_ref[...] = _mlp_core(x_ref, p0_ref, p1_ref, w1_ref, b1_ref,
                             w2_ref, b2_ref, g_ref, be_ref, eps_ref)


def _mlp_final_body(x_ref, p0_ref, p1_ref, w1_ref, b1_ref, w2_ref, b2_ref,
                    g_ref, be_ref, eps_ref, wl_ref, bl_ref, out_ref):
    hn = _mlp_core(x_ref, p0_ref, p1_ref, w1_ref, b1_ref, w2_ref, b2_ref,
                   g_ref, be_ref, eps_ref)
    hl = jnp.dot(hn, wl_ref[...], preferred_element_type=jnp.float32) + bl_ref[...]
    out_ref[...] = jnp.maximum(hl, 0.0)


_mlp = pl.pallas_call(
    _mlp_body, out_shape=jax.ShapeDtypeStruct((N, H), jnp.float32))

_mlp_final = pl.pallas_call(
    _mlp_final_body, out_shape=jax.ShapeDtypeStruct((N, H), jnp.float32))


def _final_body(xa_ref, xb_ref, wf_ref, bf_ref, out_ref):
    prod = xa_ref[...] * xb_ref[...]
    out_ref[...] = jnp.dot(prod, wf_ref[...],
                           preferred_element_type=jnp.float32) + bf_ref[...]


_final = pl.pallas_call(
    _final_body, out_shape=jax.ShapeDtypeStruct((B, OUT), jnp.float32))


def kernel(x, edge_index, ppi_pairs, idx, eps0, W1_0, b1_0, W2_0, b2_0, g0,
           be0, eps1, W1_1, b1_1, W2_1, b2_1, g1, be1, Wl, bl, Wf, bf):
    src3 = edge_index[0].reshape(NW, NCHUNK, K)
    dst3 = edge_index[1].reshape(NW, NCHUNK, K)
    zeros = jnp.zeros((N, D), jnp.float32)
    pairs_flat = ppi_pairs.reshape(-1)

    eps0_ = eps0.reshape(1, 1)
    eps1_ = eps1.reshape(1, 1)
    b1_0_, b2_0_, g0_, be0_ = (v.reshape(1, H) for v in (b1_0, b2_0, g0, be0))
    b1_1_, b2_1_, g1_, be1_ = (v.reshape(1, H) for v in (b1_1, b2_1, g1, be1))
    bl_ = bl.reshape(1, H)
    bf_ = bf.reshape(1, OUT)

    p0, p1 = _agg(x, src3, dst3, zeros)
    h0 = _mlp(x, p0, p1, W1_0, b1_0_, W2_0, b2_0_, g0_, be0_, eps0_)
    q0, q1 = _agg(h0, src3, dst3, zeros)
    hl = _mlp_final(h0, q0, q1, W1_1, b1_1_, W2_1, b2_1_, g1_, be1_,
                    eps1_, Wl, bl_)
    xa, xb = _pair_gather(hl, pairs_flat, idx)
    return _final(xa, xb, Wf, bf_)


# speedup vs baseline: 7.8982x; 1.3084x over previous
import functools

import jax
import jax.numpy as jnp
from jax import lax
from jax.experimental import pallas as pl
from jax.experimental.pallas import tpu as pltpu
from jax.experimental.pallas import tpu_sc as plsc

N = 10000
E = 320000
D = 128
H = 128
OUT = 7
P = 100000
B = 16384

NC = 2
NS = 16
NW = NC * NS

EPW = E // NW
K = 125
NCHUNK = EPW // K
NPHASE = 2
NCHP = NCHUNK // NPHASE
NBUF = 2
NOUTER = NCHP // NBUF

ROWS_T = 624

BPW = B // NW
KP = 128
NPCHUNK = BPW // KP

_MESH = plsc.VectorSubcoreMesh(
    core_axis_name="c", subcore_axis_name="s", num_cores=NC, num_subcores=NS
)


def _agg_body(x_hbm, src_hbm, dst_hbm, zeros_hbm, p0_hbm, p1_hbm,
              srcv, dstv, acc, rows0, rows1,
              gs0, gs1, ss0, ss1):
    c = lax.axis_index("c")
    s = lax.axis_index("s")
    wid = c * NS + s
    rows = (rows0, rows1)
    gsem = (gs0, gs1)
    ssem = (ss0, ss1)

    @pl.when(s == 0)
    def _():
        pltpu.sync_copy(zeros_hbm, acc)

    plsc.subcore_barrier()

    for ph in range(NPHASE):
        pltpu.sync_copy(src_hbm.at[wid, pl.ds(ph * NCHP, NCHP)], srcv)
        pltpu.sync_copy(dst_hbm.at[wid, pl.ds(ph * NCHP, NCHP)], dstv)

        for b in range(NBUF):
            pltpu.async_copy(x_hbm.at[srcv.at[b]], rows[b], gsem[b])
        for b in range(NBUF):
            pltpu.make_async_copy(x_hbm.at[srcv.at[b]], rows[b], gsem[b]).wait()
            pltpu.async_copy(rows[b], acc.at[dstv.at[b]], ssem[b], add=True)

        def outer(g, carry):
            base_i = g * NBUF
            for b in range(NBUF):
                i = base_i + b
                pltpu.make_async_copy(rows[b], acc.at[dstv.at[i]], ssem[b]).wait()
                pltpu.async_copy(x_hbm.at[srcv.at[i]], rows[b], gsem[b])
            for b in range(NBUF):
                i = base_i + b
                pltpu.make_async_copy(x_hbm.at[srcv.at[i]], rows[b], gsem[b]).wait()
                pltpu.async_copy(rows[b], acc.at[dstv.at[i]], ssem[b], add=True)
            return carry

        lax.fori_loop(1, NOUTER, outer, 0)

        for b in range(NBUF):
            pltpu.make_async_copy(rows[b], acc.at[dstv.at[b]], ssem[b]).wait()

    plsc.subcore_barrier()

    start = s * ROWS_T

    @pl.when(c == 0)
    def _():
        pltpu.sync_copy(acc.at[pl.ds(start, ROWS_T)], p0_hbm.at[pl.ds(start, ROWS_T)])

        @pl.when(s == NS - 1)
        def _():
            pltpu.sync_copy(acc.at[pl.ds(NS * ROWS_T, N - NS * ROWS_T)],
                            p0_hbm.at[pl.ds(NS * ROWS_T, N - NS * ROWS_T)])

    @pl.when(c == 1)
    def _():
        pltpu.sync_copy(acc.at[pl.ds(start, ROWS_T)], p1_hbm.at[pl.ds(start, ROWS_T)])

        @pl.when(s == NS - 1)
        def _():
            pltpu.sync_copy(acc.at[pl.ds(NS * ROWS_T, N - NS * ROWS_T)],
                            p1_hbm.at[pl.ds(NS * ROWS_T, N - NS * ROWS_T)])


_agg = functools.partial(
    pl.kernel,
    out_type=(jax.ShapeDtypeStruct((N, D), jnp.float32),
              jax.ShapeDtypeStruct((N, D), jnp.float32)),
    mesh=_MESH,
    scratch_types=(
        [pltpu.VMEM((NCHP, K), jnp.int32)] * 2
        + [pltpu.VMEM_SHARED((N, D), jnp.float32)]
        + [pltpu.VMEM((K, D), jnp.float32)] * NBUF
        + [pltpu.SemaphoreType.DMA] * (2 * NBUF)
    ),
)(_agg_body)


def _pair_body(h_hbm, pairs_hbm, idx_hbm, xa_hbm, xb_hbm,
               idxv, ia, ib, na, nb, rows_a, rows_b, sem):
    c = lax.axis_index("c")
    s = lax.axis_index("s")
    wid = c * NS + s
    base = wid * BPW

    for j in range(NPCHUNK):
        off = base + j * KP
        pltpu.sync_copy(idx_hbm.at[pl.ds(off, KP)], idxv)
        for l in range(KP // 16):
            v = idxv[pl.ds(l * 16, 16)]
            ia[pl.ds(l * 16, 16)] = v * 2
            ib[pl.ds(l * 16, 16)] = v * 2 + 1
        pltpu.async_copy(pairs_hbm.at[ia], na, sem).wait()
        pltpu.async_copy(pairs_hbm.at[ib], nb, sem).wait()
        pltpu.async_copy(h_hbm.at[na], rows_a, sem).wait()
        pltpu.async_copy(h_hbm.at[nb], rows_b, sem).wait()
        pltpu.sync_copy(rows_a, xa_hbm.at[pl.ds(off, KP)])
        pltpu.sync_copy(rows_b, xb_hbm.at[pl.ds(off, KP)])


_pair_gather = functools.partial(
    pl.kernel,
    out_type=(jax.ShapeDtypeStruct((B, D), jnp.float32),
              jax.ShapeDtypeStruct((B, D), jnp.float32)),
    mesh=_MESH,
    scratch_types=[
        pltpu.VMEM((KP,), jnp.int32),
        pltpu.VMEM((KP,), jnp.int32),
        pltpu.VMEM((KP,), jnp.int32),
        pltpu.VMEM((KP,), jnp.int32),
        pltpu.VMEM((KP,), jnp.int32),
        pltpu.VMEM((KP, D), jnp.float32),
        pltpu.VMEM((KP, D), jnp.float32),
        pltpu.SemaphoreType.DMA,
    ],
)(_pair_body)


def _mlp_core(x_ref, p0_ref, p1_ref, w1_ref, b1_ref, w2_ref, b2_ref,
              g_ref, be_ref, eps_ref):
    z = (1.0 + eps_ref[0, 0]) * x_ref[...] + p0_ref[...] + p1_ref[...]
    h = jnp.dot(z, w1_ref[...], preferred_element_type=jnp.float32) + b1_ref[...]
    h = jnp.maximum(h, 0.0)
    h = jnp.dot(h, w2_ref[...], preferred_element_type=jnp.float32) + b2_ref[...]
    h = jnp.maximum(h, 0.0)
    mean = jnp.mean(h, axis=0, keepdims=True)
    cent = h - mean
    var = jnp.mean(cent * cent, axis=0, keepdims=True)
    return cent * lax.rsqrt(var + 1e-5) * g_ref[...] + be_ref[...]


def _mlp_body(x_ref, p0_ref, p1_ref, w1_ref, b1_ref, w2_ref, b2_ref,
              g_ref, be_ref, eps_ref, out_ref):
    out_ref[...] = _mlp_core(x_ref, p0_ref, p1_ref, w1_ref, b1_ref,
                             w2_ref, b2_ref, g_ref, be_ref, eps_ref)


def _mlp_final_body(x_ref, p0_ref, p1_ref, w1_ref, b1_ref, w2_ref, b2_ref,
                    g_ref, be_ref, eps_ref, wl_ref, bl_ref, out_ref):
    hn = _mlp_core(x_ref, p0_ref, p1_ref, w1_ref, b1_ref, w2_ref, b2_ref,
                   g_ref, be_ref, eps_ref)
    hl = jnp.dot(hn, wl_ref[...], preferred_element_type=jnp.float32) + bl_ref[...]
    out_ref[...] = jnp.maximum(hl, 0.0)


_mlp = pl.pallas_call(
    _mlp_body, out_shape=jax.ShapeDtypeStruct((N, H), jnp.float32))

_mlp_final = pl.pallas_call(
    _mlp_final_body, out_shape=jax.ShapeDtypeStruct((N, H), jnp.float32))


def _final_body(xa_ref, xb_ref, wf_ref, bf_ref, out_ref):
    prod = xa_ref[...] * xb_ref[...]
    out_ref[...] = jnp.dot(prod, wf_ref[...],
                           preferred_element_type=jnp.float32) + bf_ref[...]


_final = pl.pallas_call(
    _final_body, out_shape=jax.ShapeDtypeStruct((B, OUT), jnp.float32))


def kernel(x, edge_index, ppi_pairs, idx, eps0, W1_0, b1_0, W2_0, b2_0, g0,
           be0, eps1, W1_1, b1_1, W2_1, b2_1, g1, be1, Wl, bl, Wf, bf):
    src3 = edge_index[0].reshape(NW, NCHUNK, K)
    dst3 = edge_index[1].reshape(NW, NCHUNK, K)
    zeros = jnp.zeros((N, D), jnp.float32)
    pairs_flat = ppi_pairs.reshape(-1)

    eps0_ = eps0.reshape(1, 1)
    eps1_ = eps1.reshape(1, 1)
    b1_0_, b2_0_, g0_, be0_ = (v.reshape(1, H) for v in (b1_0, b2_0, g0, be0))
    b1_1_, b2_1_, g1_, be1_ = (v.reshape(1, H) for v in (b1_1, b2_1, g1, be1))
    bl_ = bl.reshape(1, H)
    bf_ = bf.reshape(1, OUT)

    p0, p1 = _agg(x, src3, dst3, zeros)
    h0 = _mlp(x, p0, p1, W1_0, b1_0_, W2_0, b2_0_, g0_, be0_, eps0_)
    q0, q1 = _agg(h0, src3, dst3, zeros)
    hl = _mlp_final(h0, q0, q1, W1_1, b1_1_, W2_1, b2_1_, g1_, be1_,
                    eps1_, Wl, bl_)
    xa, xb = _pair_gather(hl, pairs_flat, idx)
    return _final(xa, xb, Wf, bf_)


# speedup vs baseline: 8.3737x; 1.0602x over previous
import functools

import jax
import jax.numpy as jnp
from jax import lax
from jax.experimental import pallas as pl
from jax.experimental.pallas import tpu as pltpu
from jax.experimental.pallas import tpu_sc as plsc

N = 10000
E = 320000
D = 128
H = 128
OUT = 7
P = 100000
B = 16384

NC = 2
NS = 16
NW = NC * NS

EPW = E // NW
K = 80
NCHUNK = EPW // K
NBUF = 4
NFULL = NCHUNK // NBUF

ROWS_T = 624

BPW = B // NW
KP = 128
NPCHUNK = BPW // KP

_MESH = plsc.VectorSubcoreMesh(
    core_axis_name="c", subcore_axis_name="s", num_cores=NC, num_subcores=NS
)


def _agg_body(x_hbm, src_hbm, dst_hbm, zeros_hbm, p0_hbm, p1_hbm,
              si0, si1, si2, si3, di0, di1, di2, di3,
              rows0, rows1, rows2, rows3, acc,
              is0, is1, is2, is3, gs0, gs1, gs2, gs3, ss0, ss1, ss2, ss3):
    c = lax.axis_index("c")
    s = lax.axis_index("s")
    wid = c * NS + s
    ebase = wid * EPW
    sidx = (si0, si1, si2, si3)
    didx = (di0, di1, di2, di3)
    rows = (rows0, rows1, rows2, rows3)
    isem = (is0, is1, is2, is3)
    gsem = (gs0, gs1, gs2, gs3)
    ssem = (ss0, ss1, ss2, ss3)

    @pl.when(s == 0)
    def _():
        pltpu.sync_copy(zeros_hbm, acc)

    def fire_idx(b, i):
        off = ebase + i * K
        pltpu.async_copy(src_hbm.at[pl.ds(off, K)], sidx[b], isem[b])
        pltpu.async_copy(dst_hbm.at[pl.ds(off, K)], didx[b], isem[b])

    def wait_idx(b):
        pltpu.make_async_copy(src_hbm.at[pl.ds(0, K)], sidx[b], isem[b]).wait()
        pltpu.make_async_copy(dst_hbm.at[pl.ds(0, K)], didx[b], isem[b]).wait()

    def fire_gather(b):
        pltpu.async_copy(x_hbm.at[sidx[b]], rows[b], gsem[b])

    def wait_gather(b):
        pltpu.make_async_copy(x_hbm.at[sidx[b]], rows[b], gsem[b]).wait()

    def fire_scatter(b):
        pltpu.async_copy(rows[b], acc.at[didx[b]], ssem[b], add=True)

    def wait_scatter(b):
        pltpu.make_async_copy(rows[b], acc.at[didx[b]], ssem[b]).wait()

    plsc.subcore_barrier()

    for b in range(NBUF):
        fire_idx(b, b)
    for b in range(NBUF):
        wait_idx(b)
        fire_gather(b)
    for b in range(NBUF):
        wait_gather(b)
        fire_scatter(b)

    def outer(g, carry):
        base_i = g * NBUF
        for b in range(NBUF):
            wait_scatter(b)
            fire_idx(b, base_i + b)
        for b in range(NBUF):
            wait_idx(b)
            fire_gather(b)
        for b in range(NBUF):
            wait_gather(b)
            fire_scatter(b)
        return carry

    lax.fori_loop(1, NFULL, outer, 0)

    for t, i in enumerate(range(NFULL * NBUF, NCHUNK)):
        wait_scatter(t)
        fire_idx(t, i)
        wait_idx(t)
        fire_gather(t)
        wait_gather(t)
        fire_scatter(t)

    for b in range(NBUF):
        wait_scatter(b)

    plsc.subcore_barrier()

    start = s * ROWS_T

    @pl.when(c == 0)
    def _():
        pltpu.sync_copy(acc.at[pl.ds(start, ROWS_T)], p0_hbm.at[pl.ds(start, ROWS_T)])

        @pl.when(s == NS - 1)
        def _():
            pltpu.sync_copy(acc.at[pl.ds(NS * ROWS_T, N - NS * ROWS_T)],
                            p0_hbm.at[pl.ds(NS * ROWS_T, N - NS * ROWS_T)])

    @pl.when(c == 1)
    def _():
        pltpu.sync_copy(acc.at[pl.ds(start, ROWS_T)], p1_hbm.at[pl.ds(start, ROWS_T)])

        @pl.when(s == NS - 1)
        def _():
            pltpu.sync_copy(acc.at[pl.ds(NS * ROWS_T, N - NS * ROWS_T)],
                            p1_hbm.at[pl.ds(NS * ROWS_T, N - NS * ROWS_T)])


_agg = functools.partial(
    pl.kernel,
    out_type=(jax.ShapeDtypeStruct((N, D), jnp.float32),
              jax.ShapeDtypeStruct((N, D), jnp.float32)),
    mesh=_MESH,
    scratch_types=(
        [pltpu.VMEM((K,), jnp.int32)] * (2 * NBUF)
        + [pltpu.VMEM((K, D), jnp.float32)] * NBUF
        + [pltpu.VMEM_SHARED((N, D), jnp.float32)]
        + [pltpu.SemaphoreType.DMA] * (3 * NBUF)
    ),
)(_agg_body)


def _pair_body(h_hbm, pairs_hbm, idx_hbm, xa_hbm, xb_hbm,
               idxv, ia, ib, na, nb, rows_a, rows_b, sem):
    c = lax.axis_index("c")
    s = lax.axis_index("s")
    wid = c * NS + s
    base = wid * BPW

    for j in range(NPCHUNK):
        off = base + j * KP
        pltpu.sync_copy(idx_hbm.at[pl.ds(off, KP)], idxv)
        for l in range(KP // 16):
            v = idxv[pl.ds(l * 16, 16)]
            ia[pl.ds(l * 16, 16)] = v * 2
            ib[pl.ds(l * 16, 16)] = v * 2 + 1
        pltpu.async_copy(pairs_hbm.at[ia], na, sem).wait()
        pltpu.async_copy(pairs_hbm.at[ib], nb, sem).wait()
        pltpu.async_copy(h_hbm.at[na], rows_a, sem).wait()
        pltpu.async_copy(h_hbm.at[nb], rows_b, sem).wait()
        pltpu.sync_copy(rows_a, xa_hbm.at[pl.ds(off, KP)])
        pltpu.sync_copy(rows_b, xb_hbm.at[pl.ds(off, KP)])


_pair_gather = functools.partial(
    pl.kernel,
    out_type=(jax.ShapeDtypeStruct((B, D), jnp.float32),
              jax.ShapeDtypeStruct((B, D), jnp.float32)),
    mesh=_MESH,
    scratch_types=[
        pltpu.VMEM((KP,), jnp.int32),
        pltpu.VMEM((KP,), jnp.int32),
        pltpu.VMEM((KP,), jnp.int32),
        pltpu.VMEM((KP,), jnp.int32),
        pltpu.VMEM((KP,), jnp.int32),
        pltpu.VMEM((KP, D), jnp.float32),
        pltpu.VMEM((KP, D), jnp.float32),
        pltpu.SemaphoreType.DMA,
    ],
)(_pair_body)


def _mlp_core(x_ref, p0_ref, p1_ref, w1_ref, b1_ref, w2_ref, b2_ref,
              g_ref, be_ref, eps_ref):
    z = (1.0 + eps_ref[0, 0]) * x_ref[...] + p0_ref[...] + p1_ref[...]
    h = jnp.dot(z, w1_ref[...], preferred_element_type=jnp.float32) + b1_ref[...]
    h = jnp.maximum(h, 0.0)
    h = jnp.dot(h, w2_ref[...], preferred_element_type=jnp.float32) + b2_ref[...]
    h = jnp.maximum(h, 0.0)
    mean = jnp.mean(h, axis=0, keepdims=True)
    cent = h - mean
    var = jnp.mean(cent * cent, axis=0, keepdims=True)
    return cent * lax.rsqrt(var + 1e-5) * g_ref[...] + be_ref[...]


def _mlp_body(x_ref, p0_ref, p1_ref, w1_ref, b1_ref, w2_ref, b2_ref,
              g_ref, be_ref, eps_ref, out_ref):
    out_ref[...] = _mlp_core(x_ref, p0_ref, p1_ref, w1_ref, b1_ref,
                             w2_ref, b2_ref, g_ref, be_ref, eps_ref)


def _mlp_final_body(x_ref, p0_ref, p1_ref, w1_ref, b1_ref, w2_ref, b2_ref,
                    g_ref, be_ref, eps_ref, wl_ref, bl_ref, out_ref):
    hn = _mlp_core(x_ref, p0_ref, p1_ref, w1_ref, b1_ref, w2_ref, b2_ref,
                   g_ref, be_ref, eps_ref)
    hl = jnp.dot(hn, wl_ref[...], preferred_element_type=jnp.float32) + bl_ref[...]
    out_ref[...] = jnp.maximum(hl, 0.0)


_mlp = pl.pallas_call(
    _mlp_body, out_shape=jax.ShapeDtypeStruct((N, H), jnp.float32))

_mlp_final = pl.pallas_call(
    _mlp_final_body, out_shape=jax.ShapeDtypeStruct((N, H), jnp.float32))


def _final_body(xa_ref, xb_ref, wf_ref, bf_ref, out_ref):
    prod = xa_ref[...] * xb_ref[...]
    out_ref[...] = jnp.dot(prod, wf_ref[...],
                           preferred_element_type=jnp.float32) + bf_ref[...]


_final = pl.pallas_call(
    _final_body, out_shape=jax.ShapeDtypeStruct((B, OUT), jnp.float32))


def kernel(x, edge_index, ppi_pairs, idx, eps0, W1_0, b1_0, W2_0, b2_0, g0,
           be0, eps1, W1_1, b1_1, W2_1, b2_1, g1, be1, Wl, bl, Wf, bf):
    src3 = edge_index[0]
    dst3 = edge_index[1]
    zeros = jnp.zeros((N, D), jnp.float32)
    pairs_flat = ppi_pairs.reshape(-1)

    eps0_ = eps0.reshape(1, 1)
    eps1_ = eps1.reshape(1, 1)
    b1_0_, b2_0_, g0_, be0_ = (v.reshape(1, H) for v in (b1_0, b2_0, g0, be0))
    b1_1_, b2_1_, g1_, be1_ = (v.reshape(1, H) for v in (b1_1, b2_1, g1, be1))
    bl_ = bl.reshape(1, H)
    bf_ = bf.reshape(1, OUT)

    p0, p1 = _agg(x, src3, dst3, zeros)
    h0 = _mlp(x, p0, p1, W1_0, b1_0_, W2_0, b2_0_, g0_, be0_, eps0_)
    q0, q1 = _agg(h0, src3, dst3, zeros)
    hl = _mlp_final(h0, q0, q1, W1_1, b1_1_, W2_1, b2_1_, g1_, be1_,
                    eps1_, Wl, bl_)
    xa, xb = _pair_gather(hl, pairs_flat, idx)
    return _final(xa, xb, Wf, bf_)


# speedup vs baseline: 8.6103x; 1.0283x over previous
import functools

import jax
import jax.numpy as jnp
from jax import lax
from jax.experimental import pallas as pl
from jax.experimental.pallas import tpu as pltpu
from jax.experimental.pallas import tpu_sc as plsc

N = 10000
E = 320000
D = 128
H = 128
OUT = 7
P = 100000
B = 16384

NC = 2
NS = 16
NW = NC * NS

EPW = E // NW
K = 80
NCHUNK = EPW // K
NBUF = 4
NFULL = NCHUNK // NBUF

ROWS_T = 624

BPW = B // NW
KP = 128
NPCHUNK = BPW // KP

_MESH = plsc.VectorSubcoreMesh(
    core_axis_name="c", subcore_axis_name="s", num_cores=NC, num_subcores=NS
)


def _agg_body(x_hbm, src_hbm, dst_hbm, zeros_hbm, p0_hbm, p1_hbm,
              si0, si1, si2, si3, di0, di1, di2, di3,
              rows0, rows1, rows2, rows3, acc,
              is0, is1, is2, is3, gs0, gs1, gs2, gs3, ss0, ss1, ss2, ss3):
    c = lax.axis_index("c")
    s = lax.axis_index("s")
    wid = c * NS + s
    ebase = wid * EPW
    sidx = (si0, si1, si2, si3)
    didx = (di0, di1, di2, di3)
    rows = (rows0, rows1, rows2, rows3)
    isem = (is0, is1, is2, is3)
    gsem = (gs0, gs1, gs2, gs3)
    ssem = (ss0, ss1, ss2, ss3)

    @pl.when(s == 0)
    def _():
        pltpu.sync_copy(zeros_hbm, acc)

    def fire_idx(b, i):
        off = ebase + i * K
        pltpu.async_copy(src_hbm.at[pl.ds(off, K)], sidx[b], isem[b])
        pltpu.async_copy(dst_hbm.at[pl.ds(off, K)], didx[b], isem[b])

    def wait_idx(b):
        pltpu.make_async_copy(src_hbm.at[pl.ds(0, K)], sidx[b], isem[b]).wait()
        pltpu.make_async_copy(dst_hbm.at[pl.ds(0, K)], didx[b], isem[b]).wait()

    def fire_gather(b):
        pltpu.async_copy(x_hbm.at[sidx[b]], rows[b], gsem[b])

    def wait_gather(b):
        pltpu.make_async_copy(x_hbm.at[sidx[b]], rows[b], gsem[b]).wait()

    def fire_scatter(b):
        pltpu.async_copy(rows[b], acc.at[didx[b]], ssem[b], add=True)

    def wait_scatter(b):
        pltpu.make_async_copy(rows[b], acc.at[didx[b]], ssem[b]).wait()

    plsc.subcore_barrier()

    for b in range(NBUF):
        fire_idx(b, b)
    for b in range(NBUF):
        wait_idx(b)
        fire_gather(b)
    for b in range(NBUF):
        wait_gather(b)
        fire_scatter(b)

    def outer(g, carry):
        base_i = g * NBUF
        for b in range(NBUF):
            wait_scatter(b)
            fire_idx(b, base_i + b)
        for b in range(NBUF):
            wait_idx(b)
            fire_gather(b)
        for b in range(NBUF):
            wait_gather(b)
            fire_scatter(b)
        return carry

    lax.fori_loop(1, NFULL, outer, 0)

    for t, i in enumerate(range(NFULL * NBUF, NCHUNK)):
        wait_scatter(t)
        fire_idx(t, i)
        wait_idx(t)
        fire_gather(t)
        wait_gather(t)
        fire_scatter(t)

    for b in range(NBUF):
        wait_scatter(b)

    plsc.subcore_barrier()

    start = s * ROWS_T

    @pl.when(c == 0)
    def _():
        pltpu.sync_copy(acc.at[pl.ds(start, ROWS_T)], p0_hbm.at[pl.ds(start, ROWS_T)])

        @pl.when(s == NS - 1)
        def _():
            pltpu.sync_copy(acc.at[pl.ds(NS * ROWS_T, N - NS * ROWS_T)],
                            p0_hbm.at[pl.ds(NS * ROWS_T, N - NS * ROWS_T)])

    @pl.when(c == 1)
    def _():
        pltpu.sync_copy(acc.at[pl.ds(start, ROWS_T)], p1_hbm.at[pl.ds(start, ROWS_T)])

        @pl.when(s == NS - 1)
        def _():
            pltpu.sync_copy(acc.at[pl.ds(NS * ROWS_T, N - NS * ROWS_T)],
                            p1_hbm.at[pl.ds(NS * ROWS_T, N - NS * ROWS_T)])


_agg = functools.partial(
    pl.kernel,
    out_type=(jax.ShapeDtypeStruct((N, D), jnp.float32),
              jax.ShapeDtypeStruct((N, D), jnp.float32)),
    mesh=_MESH,
    scratch_types=(
        [pltpu.VMEM((K,), jnp.int32)] * (2 * NBUF)
        + [pltpu.VMEM((K, D), jnp.float32)] * NBUF
        + [pltpu.VMEM_SHARED((N, D), jnp.float32)]
        + [pltpu.SemaphoreType.DMA] * (3 * NBUF)
    ),
)(_agg_body)


def _pair_body(h_hbm, pairs_hbm, idx_hbm, xa_hbm, xb_hbm,
               idxv, ia, ib, na, nb,
               ra0, ra1, ra2, rb0, rb1, rb2,
               pa0, pa1, pa2, pa3, pb0, pb1, pb2, pb3,
               qa0, qa1, qa2, qb0, qb1, qb2,
               wa0, wa1, wa2, wb0, wb1, wb2):
    c = lax.axis_index("c")
    s = lax.axis_index("s")
    wid = c * NS + s
    base = wid * BPW
    ra = (ra0, ra1, ra2)
    rb = (rb0, rb1, rb2)
    pa = (pa0, pa1, pa2, pa3)
    pb = (pb0, pb1, pb2, pb3)
    qa = (qa0, qa1, qa2)
    qb = (qb0, qb1, qb2)
    wa = (wa0, wa1, wa2)
    wb = (wb0, wb1, wb2)

    pltpu.sync_copy(idx_hbm.at[pl.ds(base, BPW)], idxv)
    for l in range(BPW // 16):
        v = idxv[pl.ds(l * 16, 16)]
        ia[pl.ds(l * 16, 16)] = v * 2
        ib[pl.ds(l * 16, 16)] = v * 2 + 1

    def sl(j):
        return pl.ds(j * KP, KP)

    def off(j):
        return pl.ds(base + j * KP, KP)

    def fire_ids(j):
        pltpu.async_copy(pairs_hbm.at[ia.at[sl(j)]], na.at[sl(j)], pa[j])
        pltpu.async_copy(pairs_hbm.at[ib.at[sl(j)]], nb.at[sl(j)], pb[j])

    def fire_rows(j):
        b = j % 3
        pltpu.make_async_copy(pairs_hbm.at[ia.at[sl(j)]], na.at[sl(j)], pa[j]).wait()
        pltpu.async_copy(h_hbm.at[na.at[sl(j)]], ra[b], qa[b])
        pltpu.make_async_copy(pairs_hbm.at[ib.at[sl(j)]], nb.at[sl(j)], pb[j]).wait()
        pltpu.async_copy(h_hbm.at[nb.at[sl(j)]], rb[b], qb[b])

    def fire_wb(j):
        b = j % 3
        pltpu.make_async_copy(h_hbm.at[na.at[sl(j)]], ra[b], qa[b]).wait()
        pltpu.async_copy(ra[b], xa_hbm.at[off(j)], wa[b])
        pltpu.make_async_copy(h_hbm.at[nb.at[sl(j)]], rb[b], qb[b]).wait()
        pltpu.async_copy(rb[b], xb_hbm.at[off(j)], wb[b])

    def wait_wb(j):
        b = j % 3
        pltpu.make_async_copy(ra[b], xa_hbm.at[off(j)], wa[b]).wait()
        pltpu.make_async_copy(rb[b], xb_hbm.at[off(j)], wb[b]).wait()

    for j in range(NPCHUNK):
        fire_ids(j)
    fire_rows(0)
    fire_rows(1)
    fire_rows(2)
    fire_wb(0)
    wait_wb(0)
    fire_rows(3)
    fire_wb(1)
    fire_wb(2)
    fire_wb(3)
    wait_wb(1)
    wait_wb(2)
    wait_wb(3)


_pair_gather = functools.partial(
    pl.kernel,
    out_type=(jax.ShapeDtypeStruct((B, D), jnp.float32),
              jax.ShapeDtypeStruct((B, D), jnp.float32)),
    mesh=_MESH,
    scratch_types=(
        [pltpu.VMEM((BPW,), jnp.int32)] * 5
        + [pltpu.VMEM((KP, D), jnp.float32)] * 6
        + [pltpu.SemaphoreType.DMA] * 20
    ),
)(_pair_body)


def _mlp_core(x_ref, p0_ref, p1_ref, w1_ref, b1_ref, w2_ref, b2_ref,
              g_ref, be_ref, eps_ref):
    z = ((1.0 + eps_ref[0, 0]) * x_ref[...]
         + p0_ref[...].astype(jnp.float32) + p1_ref[...].astype(jnp.float32))
    h = jnp.dot(z, w1_ref[...], preferred_element_type=jnp.float32) + b1_ref[...]
    h = jnp.maximum(h, 0.0)
    h = jnp.dot(h, w2_ref[...], preferred_element_type=jnp.float32) + b2_ref[...]
    h = jnp.maximum(h, 0.0)
    mean = jnp.mean(h, axis=0, keepdims=True)
    cent = h - mean
    var = jnp.mean(cent * cent, axis=0, keepdims=True)
    return cent * lax.rsqrt(var + 1e-5) * g_ref[...] + be_ref[...]


def _mlp_body(x_ref, p0_ref, p1_ref, w1_ref, b1_ref, w2_ref, b2_ref,
              g_ref, be_ref, eps_ref, out_ref):
    out_ref[...] = _mlp_core(x_ref, p0_ref, p1_ref, w1_ref, b1_ref,
                             w2_ref, b2_ref, g_ref, be_ref, eps_ref)


def _mlp_final_body(x_ref, p0_ref, p1_ref, w1_ref, b1_ref, w2_ref, b2_ref,
                    g_ref, be_ref, eps_ref, wl_ref, bl_ref, out_ref):
    hn = _mlp_core(x_ref, p0_ref, p1_ref, w1_ref, b1_ref, w2_ref, b2_ref,
                   g_ref, be_ref, eps_ref)
    hl = jnp.dot(hn, wl_ref[...], preferred_element_type=jnp.float32) + bl_ref[...]
    out_ref[...] = jnp.maximum(hl, 0.0)


_mlp = pl.pallas_call(
    _mlp_body, out_shape=jax.ShapeDtypeStruct((N, H), jnp.float32))

_mlp_final = pl.pallas_call(
    _mlp_final_body, out_shape=jax.ShapeDtypeStruct((N, H), jnp.float32))


def _final_body(xa_ref, xb_ref, wf_ref, bf_ref, out_ref):
    prod = xa_ref[...] * xb_ref[...]
    out_ref[...] = jnp.dot(prod, wf_ref[...],
                           preferred_element_type=jnp.float32) + bf_ref[...]


_final = pl.pallas_call(
    _final_body, out_shape=jax.ShapeDtypeStruct((B, OUT), jnp.float32))


def kernel(x, edge_index, ppi_pairs, idx, eps0, W1_0, b1_0, W2_0, b2_0, g0,
           be0, eps1, W1_1, b1_1, W2_1, b2_1, g1, be1, Wl, bl, Wf, bf):
    src3 = edge_index[0]
    dst3 = edge_index[1]
    zeros = jnp.zeros((N, D), jnp.float32)
    pairs_flat = ppi_pairs.reshape(-1)

    eps0_ = eps0.reshape(1, 1)
    eps1_ = eps1.reshape(1, 1)
    b1_0_, b2_0_, g0_, be0_ = (v.reshape(1, H) for v in (b1_0, b2_0, g0, be0))
    b1_1_, b2_1_, g1_, be1_ = (v.reshape(1, H) for v in (b1_1, b2_1, g1, be1))
    bl_ = bl.reshape(1, H)
    bf_ = bf.reshape(1, OUT)

    p0, p1 = _agg(x, src3, dst3, zeros)
    h0 = _mlp(x, p0, p1, W1_0, b1_0_, W2_0, b2_0_, g0_, be0_, eps0_)
    q0, q1 = _agg(h0, src3, dst3, zeros)
    hl = _mlp_final(h0, q0, q1, W1_1, b1_1_, W2_1, b2_1_, g1_, be1_,
                    eps1_, Wl, bl_)
    xa, xb = _pair_gather(hl, pairs_flat, idx)
    return _final(xa, xb, Wf, bf_)


# speedup vs baseline: 8.8155x; 1.0238x over previous
import functools

import jax
import jax.numpy as jnp
from jax import lax
from jax.experimental import pallas as pl
from jax.experimental.pallas import tpu as pltpu
from jax.experimental.pallas import tpu_sc as plsc

N = 10000
E = 320000
D = 128
H = 128
OUT = 7
P = 100000
B = 16384

NC = 2
NS = 16
NW = NC * NS

EPW = E // NW
K = 40
NCHUNK = EPW // K
NBUF = 8
NFULL = NCHUNK // NBUF

ROWS_T = 624

BPW = B // NW
KP = 128
NPCHUNK = BPW // KP

_MESH = plsc.VectorSubcoreMesh(
    core_axis_name="c", subcore_axis_name="s", num_cores=NC, num_subcores=NS
)


def _agg_body(x_hbm, src_hbm, dst_hbm, zeros_hbm, p0_hbm, p1_hbm,
              si0, si1, si2, si3, si4, si5, si6, si7,
              di0, di1, di2, di3, di4, di5, di6, di7,
              rows0, rows1, rows2, rows3, rows4, rows5, rows6, rows7, acc,
              is0, is1, is2, is3, is4, is5, is6, is7,
              gs0, gs1, gs2, gs3, gs4, gs5, gs6, gs7,
              ss0, ss1, ss2, ss3, ss4, ss5, ss6, ss7):
    c = lax.axis_index("c")
    s = lax.axis_index("s")
    wid = c * NS + s
    ebase = wid * EPW
    sidx = (si0, si1, si2, si3, si4, si5, si6, si7)
    didx = (di0, di1, di2, di3, di4, di5, di6, di7)
    rows = (rows0, rows1, rows2, rows3, rows4, rows5, rows6, rows7)
    isem = (is0, is1, is2, is3, is4, is5, is6, is7)
    gsem = (gs0, gs1, gs2, gs3, gs4, gs5, gs6, gs7)
    ssem = (ss0, ss1, ss2, ss3, ss4, ss5, ss6, ss7)

    @pl.when(s == 0)
    def _():
        pltpu.sync_copy(zeros_hbm, acc)

    def fire_idx(b, i):
        off = ebase + i * K
        pltpu.async_copy(src_hbm.at[pl.ds(off, K)], sidx[b], isem[b])
        pltpu.async_copy(dst_hbm.at[pl.ds(off, K)], didx[b], isem[b])

    def wait_idx(b):
        pltpu.make_async_copy(src_hbm.at[pl.ds(0, K)], sidx[b], isem[b]).wait()
        pltpu.make_async_copy(dst_hbm.at[pl.ds(0, K)], didx[b], isem[b]).wait()

    def fire_gather(b):
        pltpu.async_copy(x_hbm.at[sidx[b]], rows[b], gsem[b])

    def wait_gather(b):
        pltpu.make_async_copy(x_hbm.at[sidx[b]], rows[b], gsem[b]).wait()

    def fire_scatter(b):
        pltpu.async_copy(rows[b], acc.at[didx[b]], ssem[b], add=True)

    def wait_scatter(b):
        pltpu.make_async_copy(rows[b], acc.at[didx[b]], ssem[b]).wait()

    plsc.subcore_barrier()

    for b in range(NBUF):
        fire_idx(b, b)
    for b in range(NBUF):
        wait_idx(b)
        fire_gather(b)
    for b in range(NBUF):
        wait_gather(b)
        fire_scatter(b)

    def outer(g, carry):
        base_i = g * NBUF
        for b in range(NBUF):
            wait_scatter(b)
            fire_idx(b, base_i + b)
        for b in range(NBUF):
            wait_idx(b)
            fire_gather(b)
        for b in range(NBUF):
            wait_gather(b)
            fire_scatter(b)
        return carry

    lax.fori_loop(1, NFULL, outer, 0)

    for t, i in enumerate(range(NFULL * NBUF, NCHUNK)):
        wait_scatter(t)
        fire_idx(t, i)
        wait_idx(t)
        fire_gather(t)
        wait_gather(t)
        fire_scatter(t)

    for b in range(NBUF):
        wait_scatter(b)

    plsc.subcore_barrier()

    start = s * ROWS_T

    @pl.when(c == 0)
    def _():
        pltpu.sync_copy(acc.at[pl.ds(start, ROWS_T)], p0_hbm.at[pl.ds(start, ROWS_T)])

        @pl.when(s == NS - 1)
        def _():
            pltpu.sync_copy(acc.at[pl.ds(NS * ROWS_T, N - NS * ROWS_T)],
                            p0_hbm.at[pl.ds(NS * ROWS_T, N - NS * ROWS_T)])

    @pl.when(c == 1)
    def _():
        pltpu.sync_copy(acc.at[pl.ds(start, ROWS_T)], p1_hbm.at[pl.ds(start, ROWS_T)])

        @pl.when(s == NS - 1)
        def _():
            pltpu.sync_copy(acc.at[pl.ds(NS * ROWS_T, N - NS * ROWS_T)],
                            p1_hbm.at[pl.ds(NS * ROWS_T, N - NS * ROWS_T)])


_agg = functools.partial(
    pl.kernel,
    out_type=(jax.ShapeDtypeStruct((N, D), jnp.float32),
              jax.ShapeDtypeStruct((N, D), jnp.float32)),
    mesh=_MESH,
    scratch_types=(
        [pltpu.VMEM((K,), jnp.int32)] * (2 * NBUF)
        + [pltpu.VMEM((K, D), jnp.float32)] * NBUF
        + [pltpu.VMEM_SHARED((N, D), jnp.float32)]
        + [pltpu.SemaphoreType.DMA] * (3 * NBUF)
    ),
)(_agg_body)


def _pair_body(h_hbm, pairs_hbm, idx_hbm, xa_hbm, xb_hbm,
               idxv, ia, ib, na, nb,
               ra0, ra1, ra2, rb0, rb1, rb2,
               pa0, pa1, pa2, pa3, pb0, pb1, pb2, pb3,
               qa0, qa1, qa2, qb0, qb1, qb2,
               wa0, wa1, wa2, wb0, wb1, wb2):
    c = lax.axis_index("c")
    s = lax.axis_index("s")
    wid = c * NS + s
    base = wid * BPW
    ra = (ra0, ra1, ra2)
    rb = (rb0, rb1, rb2)
    pa = (pa0, pa1, pa2, pa3)
    pb = (pb0, pb1, pb2, pb3)
    qa = (qa0, qa1, qa2)
    qb = (qb0, qb1, qb2)
    wa = (wa0, wa1, wa2)
    wb = (wb0, wb1, wb2)

    pltpu.sync_copy(idx_hbm.at[pl.ds(base, BPW)], idxv)
    for l in range(BPW // 16):
        v = idxv[pl.ds(l * 16, 16)]
        ia[pl.ds(l * 16, 16)] = v * 2
        ib[pl.ds(l * 16, 16)] = v * 2 + 1

    def sl(j):
        return pl.ds(j * KP, KP)

    def off(j):
        return pl.ds(base + j * KP, KP)

    def fire_ids(j):
        pltpu.async_copy(pairs_hbm.at[ia.at[sl(j)]], na.at[sl(j)], pa[j])
        pltpu.async_copy(pairs_hbm.at[ib.at[sl(j)]], nb.at[sl(j)], pb[j])

    def fire_rows(j):
        b = j % 3
        pltpu.make_async_copy(pairs_hbm.at[ia.at[sl(j)]], na.at[sl(j)], pa[j]).wait()
        pltpu.async_copy(h_hbm.at[na.at[sl(j)]], ra[b], qa[b])
        pltpu.make_async_copy(pairs_hbm.at[ib.at[sl(j)]], nb.at[sl(j)], pb[j]).wait()
        pltpu.async_copy(h_hbm.at[nb.at[sl(j)]], rb[b], qb[b])

    def fire_wb(j):
        b = j % 3
        pltpu.make_async_copy(h_hbm.at[na.at[sl(j)]], ra[b], qa[b]).wait()
        pltpu.async_copy(ra[b], xa_hbm.at[off(j)], wa[b])
        pltpu.make_async_copy(h_hbm.at[nb.at[sl(j)]], rb[b], qb[b]).wait()
        pltpu.async_copy(rb[b], xb_hbm.at[off(j)], wb[b])

    def wait_wb(j):
        b = j % 3
        pltpu.make_async_copy(ra[b], xa_hbm.at[off(j)], wa[b]).wait()
        pltpu.make_async_copy(rb[b], xb_hbm.at[off(j)], wb[b]).wait()

    for j in range(NPCHUNK):
        fire_ids(j)
    fire_rows(0)
    fire_rows(1)
    fire_rows(2)
    fire_wb(0)
    wait_wb(0)
    fire_rows(3)
    fire_wb(1)
    fire_wb(2)
    fire_wb(3)
    wait_wb(1)
    wait_wb(2)
    wait_wb(3)


_pair_gather = functools.partial(
    pl.kernel,
    out_type=(jax.ShapeDtypeStruct((B, D), jnp.float32),
              jax.ShapeDtypeStruct((B, D), jnp.float32)),
    mesh=_MESH,
    scratch_types=(
        [pltpu.VMEM((BPW,), jnp.int32)] * 5
        + [pltpu.VMEM((KP, D), jnp.float32)] * 6
        + [pltpu.SemaphoreType.DMA] * 20
    ),
)(_pair_body)


def _mlp_core(x_ref, p0_ref, p1_ref, w1_ref, b1_ref, w2_ref, b2_ref,
              g_ref, be_ref, eps_ref):
    z = ((1.0 + eps_ref[0, 0]) * x_ref[...]
         + p0_ref[...].astype(jnp.float32) + p1_ref[...].astype(jnp.float32))
    h = jnp.dot(z, w1_ref[...], preferred_element_type=jnp.float32) + b1_ref[...]
    h = jnp.maximum(h, 0.0)
    h = jnp.dot(h, w2_ref[...], preferred_element_type=jnp.float32) + b2_ref[...]
    h = jnp.maximum(h, 0.0)
    mean = jnp.mean(h, axis=0, keepdims=True)
    cent = h - mean
    var = jnp.mean(cent * cent, axis=0, keepdims=True)
    return cent * lax.rsqrt(var + 1e-5) * g_ref[...] + be_ref[...]


def _mlp_body(x_ref, p0_ref, p1_ref, w1_ref, b1_ref, w2_ref, b2_ref,
              g_ref, be_ref, eps_ref, out_ref):
    out_ref[...] = _mlp_core(x_ref, p0_ref, p1_ref, w1_ref, b1_ref,
                             w2_ref, b2_ref, g_ref, be_ref, eps_ref)


def _mlp_final_body(x_ref, p0_ref, p1_ref, w1_ref, b1_ref, w2_ref, b2_ref,
                    g_ref, be_ref, eps_ref, wl_ref, bl_ref, out_ref):
    hn = _mlp_core(x_ref, p0_ref, p1_ref, w1_ref, b1_ref, w2_ref, b2_ref,
                   g_ref, be_ref, eps_ref)
    hl = jnp.dot(hn, wl_ref[...], preferred_element_type=jnp.float32) + bl_ref[...]
    out_ref[...] = jnp.maximum(hl, 0.0)


_mlp = pl.pallas_call(
    _mlp_body, out_shape=jax.ShapeDtypeStruct((N, H), jnp.float32))

_mlp_final = pl.pallas_call(
    _mlp_final_body, out_shape=jax.ShapeDtypeStruct((N, H), jnp.float32))


def _final_body(xa_ref, xb_ref, wf_ref, bf_ref, out_ref):
    prod = xa_ref[...] * xb_ref[...]
    out_ref[...] = jnp.dot(prod, wf_ref[...],
                           preferred_element_type=jnp.float32) + bf_ref[...]


_final = pl.pallas_call(
    _final_body, out_shape=jax.ShapeDtypeStruct((B, OUT), jnp.float32))


def kernel(x, edge_index, ppi_pairs, idx, eps0, W1_0, b1_0, W2_0, b2_0, g0,
           be0, eps1, W1_1, b1_1, W2_1, b2_1, g1, be1, Wl, bl, Wf, bf):
    src3 = edge_index[0]
    dst3 = edge_index[1]
    zeros = jnp.zeros((N, D), jnp.float32)
    pairs_flat = ppi_pairs.reshape(-1)

    eps0_ = eps0.reshape(1, 1)
    eps1_ = eps1.reshape(1, 1)
    b1_0_, b2_0_, g0_, be0_ = (v.reshape(1, H) for v in (b1_0, b2_0, g0, be0))
    b1_1_, b2_1_, g1_, be1_ = (v.reshape(1, H) for v in (b1_1, b2_1, g1, be1))
    bl_ = bl.reshape(1, H)
    bf_ = bf.reshape(1, OUT)

    p0, p1 = _agg(x, src3, dst3, zeros)
    h0 = _mlp(x, p0, p1, W1_0, b1_0_, W2_0, b2_0_, g0_, be0_, eps0_)
    q0, q1 = _agg(h0, src3, dst3, zeros)
    hl = _mlp_final(h0, q0, q1, W1_1, b1_1_, W2_1, b2_1_, g1_, be1_,
                    eps1_, Wl, bl_)
    xa, xb = _pair_gather(hl, pairs_flat, idx)
    return _final(xa, xb, Wf, bf_)
